# Initial kernel scaffold; baseline (speedup 1.0000x reference)
import jax
import jax.numpy as jnp
from jax.experimental import pallas as pl


def kernel(x, saliency, Wc, bc, Ww, bw, Wo, bo):
    raise NotImplementedError("write your pallas kernel here")


# speedup vs baseline: 1.2943x; 1.2943x over previous
import jax
import jax.numpy as jnp
from jax import lax
from jax.experimental import pallas as pl
from jax.experimental.pallas import tpu as pltpu

N_CTX = 20000
TILE = 512
N_PAD = 20480
GRID = N_PAD // TILE
K = 100
KPAD = 128
UNIT = 2.0


def _heads_kernel(x_ref, sal_ref, w_ref, b_ref, c_ref, wv_ref, ov_ref):
    i = pl.program_id(0)
    xb = x_ref[...]
    w = w_ref[...]
    y = lax.dot_general(xb, w, (((2,), (0,)), ((), ())),
                        preferred_element_type=jnp.float32)
    y = y + b_ref[...][:, None, :]
    col = i * TILE + lax.broadcasted_iota(jnp.int32, (8, TILE), 1)
    valid = col < N_CTX
    mask = jnp.where(sal_ref[...] >= 0, 1.0, 0.0)
    c = jax.nn.sigmoid(y[..., 0]) * mask
    c_ref[...] = jnp.where(valid, c, 0.0)
    wv_ref[...] = jnp.where(valid, y[..., 1], 0.0)
    ov_ref[...] = jnp.where(valid, y[..., 2], 0.0)


def _decode_kernel(c_ref, w_ref, o_ref, left_ref, right_ref, score_ref,
                   kept_ref):
    c = c_ref[...]
    colN = lax.broadcasted_iota(jnp.int32, (8, N_PAD), 1)
    r = pltpu.roll(c, shift=N_PAD - 1, axis=1)
    l = pltpu.roll(c, shift=1, axis=1)
    r = jnp.where(colN == N_PAD - 1, 0.0, r)
    l = jnp.where(colN == 0, 0.0, l)
    hmax = jnp.maximum(c, jnp.maximum(l, r))
    kept_ref[...] = jnp.where(hmax == c, c, 0.0)

    lane = lax.broadcasted_iota(jnp.int32, (8, KPAD), 1)

    def body(i, carry):
        sc, idf, wv, ov = carry
        kept = kept_ref[...]
        m = jnp.max(kept, axis=1)
        ism = kept == m[:, None]
        idx = jnp.min(jnp.where(ism, colN, jnp.int32(1 << 30)), axis=1)
        sel = colN == idx[:, None]
        wsel = jnp.sum(jnp.where(sel, w_ref[...], 0.0), axis=1)
        osel = jnp.sum(jnp.where(sel, o_ref[...], 0.0), axis=1)
        kept_ref[...] = jnp.where(sel, -1.0, kept)
        here = lane == i
        sc = jnp.where(here, m[:, None], sc)
        idf = jnp.where(here, idx.astype(jnp.float32)[:, None], idf)
        wv = jnp.where(here, wsel[:, None], wv)
        ov = jnp.where(here, osel[:, None], ov)
        return sc, idf, wv, ov

    z = jnp.zeros((8, KPAD), jnp.float32)
    sc, idf, wv, ov = lax.fori_loop(0, K, body, (z, z, z, z))

    off = jnp.maximum(ov, 0.0)
    win = jnp.maximum(wv, 0.0)
    center = idf + off
    left = jnp.clip(center - win / 2.0, 0.0, N_CTX - 1.0) * UNIT
    right = jnp.clip(center + win / 2.0, 0.0, N_CTX - 1.0) * UNIT + UNIT
    left_ref[...] = left[:, :K]
    right_ref[...] = right[:, :K]
    score_ref[...] = sc[:, :K]


@jax.jit
def kernel(x, saliency, Wc, bc, Ww, bw, Wo, bo):
    w = jnp.concatenate([Wc, Ww, Wo], axis=1)
    b = jnp.broadcast_to(jnp.stack([bc[0], bw[0], bo[0]])[None, :], (8, 3))
    c, wv, ov = pl.pallas_call(
        _heads_kernel,
        grid=(GRID,),
        in_specs=[
            pl.BlockSpec((8, TILE, 256), lambda i: (0, i, 0)),
            pl.BlockSpec((8, TILE), lambda i: (0, i)),
            pl.BlockSpec((256, 3), lambda i: (0, 0)),
            pl.BlockSpec((8, 3), lambda i: (0, 0)),
        ],
        out_specs=[pl.BlockSpec((8, TILE), lambda i: (0, i))] * 3,
        out_shape=[jax.ShapeDtypeStruct((8, N_PAD), jnp.float32)] * 3,
    )(x, saliency, w, b)
    left, right, score = pl.pallas_call(
        _decode_kernel,
        out_shape=[jax.ShapeDtypeStruct((8, K), jnp.float32)] * 3,
        scratch_shapes=[pltpu.VMEM((8, N_PAD), jnp.float32)],
    )(c, wv, ov)
    return jnp.stack([left, right, score], axis=2)
